# baseline (speedup 1.0000x reference)
import functools

import jax
import jax.numpy as jnp
from jax import lax
from jax.experimental import pallas as pl
from jax.experimental.pallas import tpu as pltpu
from jax.experimental.pallas import tpu_sc as plsc

N = 10000
E = 320000
DF = 128
H = 256
ED = 128

NC = 2
NS = 16
NW = NC * NS
FPW = H // NW
FPP = 4
NPASS = FPW // FPP

CHK = 4000
NCHK = E // CHK
GRP = CHK // 16
UNROLL = 5

DCH = 80
DROW = E // NW // DCH
NP_DEG = 10240

_mesh = plsc.VectorSubcoreMesh(
    core_axis_name="c", subcore_axis_name="s", num_cores=NC, num_subcores=NS)


@functools.partial(
    pl.kernel,
    out_type=jax.ShapeDtypeStruct((NC, NP_DEG), jnp.float32),
    mesh=_mesh,
    scratch_types=[
        pltpu.VMEM((DROW, DCH), jnp.int32),
        pltpu.VMEM((DCH,), jnp.float32),
        pltpu.VMEM((640,), jnp.float32),
        pltpu.VMEM_SHARED((NP_DEG,), jnp.float32),
    ],
)
def _sc_degree(dst_h, out_h, dstv, onesv, zv, acc):
    c = lax.axis_index("c")
    s = lax.axis_index("s")
    w = s * NC + c
    pltpu.sync_copy(dst_h.at[w], dstv)

    def fill_ones(i, _):
        onesv[pl.ds(i * 16, 16)] = jnp.ones((16,), jnp.float32)
        return 0
    lax.fori_loop(0, DCH // 16, fill_ones, 0)

    def fill_zero(i, _):
        zv[pl.ds(i * 16, 16)] = jnp.zeros((16,), jnp.float32)
        return 0
    lax.fori_loop(0, 640 // 16, fill_zero, 0)

    pltpu.sync_copy(zv, acc.at[pl.ds(s * 640, 640)])
    plsc.subcore_barrier()

    def body(r, _):
        pltpu.sync_copy(onesv, acc.at[dstv.at[r]], add=True)
        return 0
    lax.fori_loop(0, DROW, body, 0)

    plsc.subcore_barrier()
    pltpu.sync_copy(acc.at[pl.ds(s * 640, 640)], out_h.at[c, pl.ds(s * 640, 640)])


@functools.partial(
    pl.kernel,
    out_type=jax.ShapeDtypeStruct((H * N,), jnp.float32),
    mesh=_mesh,
    compiler_params=pltpu.CompilerParams(needs_layout_passes=False),
    scratch_types=[
        [pltpu.VMEM((N,), jnp.float32) for _ in range(FPP)],
        [pltpu.VMEM((N,), jnp.float32) for _ in range(FPP)],
        [pltpu.VMEM((CHK,), jnp.int32) for _ in range(2)],
        [pltpu.VMEM((CHK,), jnp.int32) for _ in range(2)],
        pltpu.SemaphoreType.DMA,
        pltpu.SemaphoreType.DMA,
    ],
)
def _sc_scatter(ysf_h, src_h, dst_h, accf_h, ysr, acr, sb, db, sem0, sem1):
    c = lax.axis_index("c")
    s = lax.axis_index("s")
    w = s * NC + c

    def load_idx(j, b, sem):
        ds = pltpu.async_copy(src_h.at[pl.ds(j * CHK, CHK)], sb[b], sem)
        dd = pltpu.async_copy(dst_h.at[pl.ds(j * CHK, CHK)], db[b], sem)
        return ds, dd

    def wait_idx(j, b, sem):
        pltpu.make_async_copy(src_h.at[pl.ds(j * CHK, CHK)], sb[b], sem).wait()
        pltpu.make_async_copy(dst_h.at[pl.ds(j * CHK, CHK)], db[b], sem).wait()

    def compute(b):
        def gathers(kk):
            dl, vl = [], []
            for u in range(UNROLL):
                k = kk * UNROLL + u
                s16 = sb[b][pl.ds(k * 16, 16)]
                dl.append(db[b][pl.ds(k * 16, 16)])
                vl.append(tuple(plsc.load_gather(ysr[q], [s16])
                                for q in range(FPP)))
            return tuple(dl), tuple(vl)

        def scatters(dl, vl):
            for u in range(UNROLL):
                for q in range(FPP):
                    plsc.addupdate_scatter(acr[q], [dl[u]], vl[u][q])

        def body(kk, carry):
            nxt = gathers(kk)
            scatters(*carry)
            return nxt
        last = lax.fori_loop(1, GRP // UNROLL, body, gathers(0))
        scatters(*last)

    for p in range(NPASS):
        base = (w * FPW + p * FPP) * N
        for q in range(FPP):
            pltpu.sync_copy(ysf_h.at[pl.ds(base + q * N, N)], ysr[q])
            pltpu.sync_copy(ysf_h.at[pl.ds(base + q * N, N)], acr[q])

        load_idx(0, 0, sem0)

        def outer(jj, _):
            j0 = 2 * jj
            load_idx(j0 + 1, 1, sem1)
            wait_idx(j0, 0, sem0)
            compute(0)

            @pl.when(jj < NCHK // 2 - 1)
            def _():
                load_idx(j0 + 2, 0, sem0)

            wait_idx(j0 + 1, 1, sem1)
            compute(1)
            return 0
        lax.fori_loop(0, NCHK // 2, outer, 0)

        for q in range(FPP):
            pltpu.sync_copy(acr[q], accf_h.at[pl.ds(base + q * N, N)])


def _k0_body(xt_ref, deg2_ref, w1t_ref, b1_ref, w2t_ref, b2_ref, w0t_ref,
             yst_ref, dinv_ref):
    deg = deg2_ref[0:1, :] + deg2_ref[1:2, :] + 1.0
    dinv = lax.rsqrt(deg)
    dinv_ref[...] = dinv
    h = jnp.maximum(
        jnp.dot(w1t_ref[...], xt_ref[...], preferred_element_type=jnp.float32)
        + b1_ref[...], 0.0)
    h = jnp.dot(w2t_ref[...], h, preferred_element_type=jnp.float32) + b2_ref[...]
    y = jnp.dot(w0t_ref[...], h, preferred_element_type=jnp.float32)
    yst_ref[...] = y * dinv


_k0_call = pl.pallas_call(
    _k0_body,
    out_shape=[
        jax.ShapeDtypeStruct((H, N), jnp.float32),
        jax.ShapeDtypeStruct((1, N), jnp.float32),
    ],
)


def _mid_body(at_ref, dinv_ref, bprev_ref, wnt_ref, yst_ref):
    dinv = dinv_ref[...]
    h = jnp.maximum(dinv * at_ref[...] + bprev_ref[...], 0.0)
    y = jnp.dot(wnt_ref[...], h, preferred_element_type=jnp.float32)
    yst_ref[...] = y * dinv


_mid_call = pl.pallas_call(
    _mid_body,
    out_shape=jax.ShapeDtypeStruct((H, N), jnp.float32),
)


def _head_body(at_ref, dinv_ref, b2_ref,
               muW1t_ref, mub1_ref, muW2t_ref, mub2_ref,
               lvW1t_ref, lvb1_ref, lvW2t_ref, lvb2_ref,
               z_ref, mu_ref, lv_ref):
    h = jnp.maximum(dinv_ref[...] * at_ref[...] + b2_ref[...], 0.0)
    mean = jnp.sum(h, axis=1, keepdims=True) * (1.0 / N)
    mx = jnp.max(h, axis=1, keepdims=True)
    g = jnp.concatenate([mean, mx], axis=0)
    mu = jnp.maximum(
        jnp.dot(muW1t_ref[...], g, preferred_element_type=jnp.float32)
        + mub1_ref[...], 0.0)
    mu = jnp.dot(muW2t_ref[...], mu, preferred_element_type=jnp.float32) + mub2_ref[...]
    lv = jnp.maximum(
        jnp.dot(lvW1t_ref[...], g, preferred_element_type=jnp.float32)
        + lvb1_ref[...], 0.0)
    lv = jnp.dot(lvW2t_ref[...], lv, preferred_element_type=jnp.float32) + lvb2_ref[...]
    z_ref[...] = mu
    mu_ref[...] = mu
    lv_ref[...] = lv


_head_call = pl.pallas_call(
    _head_body,
    out_shape=[
        jax.ShapeDtypeStruct((ED, 1), jnp.float32),
        jax.ShapeDtypeStruct((ED, 1), jnp.float32),
        jax.ShapeDtypeStruct((ED, 1), jnp.float32),
    ],
)


def kernel(x, edge_index, enc_W1, enc_b1, enc_W2, enc_b2,
           conv_W0, conv_b0, conv_W1, conv_b1, conv_W2, conv_b2,
           mu_W1, mu_b1, mu_W2, mu_b2, lv_W1, lv_b1, lv_W2, lv_b2):
    src = edge_index[0]
    dst = edge_index[1]
    dstd = dst.reshape(NW, DROW, DCH)

    degp = _sc_degree(dstd)
    deg2 = degp[:, :N]

    yst, dinv = _k0_call(x.T, deg2, enc_W1.T, enc_b1.reshape(H, 1),
                         enc_W2.T, enc_b2.reshape(H, 1), conv_W0.T)
    at = _sc_scatter(yst.reshape(H * N), src, dst).reshape(H, N)
    yst = _mid_call(at, dinv, conv_b0.reshape(H, 1), conv_W1.T)
    at = _sc_scatter(yst.reshape(H * N), src, dst).reshape(H, N)
    yst = _mid_call(at, dinv, conv_b1.reshape(H, 1), conv_W2.T)
    at = _sc_scatter(yst.reshape(H * N), src, dst).reshape(H, N)
    z, mu, lv = _head_call(at, dinv, conv_b2.reshape(H, 1),
                           mu_W1.T, mu_b1.reshape(H, 1), mu_W2.T, mu_b2.reshape(ED, 1),
                           lv_W1.T, lv_b1.reshape(H, 1), lv_W2.T, lv_b2.reshape(ED, 1))
    return (z.reshape(1, ED), mu.reshape(1, ED), lv.reshape(1, ED))


# speedup vs baseline: 1.2102x; 1.2102x over previous
import functools

import jax
import jax.numpy as jnp
from jax import lax
from jax.experimental import pallas as pl
from jax.experimental.pallas import tpu as pltpu
from jax.experimental.pallas import tpu_sc as plsc

N = 10000
E = 320000
DF = 128
H = 256
ED = 128

NC = 2
NS = 16
NW = NC * NS
FPW = H // NW
PK = FPW // 2

CHK = 2000
NCHK = E // CHK
GRP = CHK // 16
UNROLL = 5

DCH = 80
DROW = E // NW // DCH
NP_DEG = 10240

_mesh = plsc.VectorSubcoreMesh(
    core_axis_name="c", subcore_axis_name="s", num_cores=NC, num_subcores=NS)


@functools.partial(
    pl.kernel,
    out_type=jax.ShapeDtypeStruct((NC, NP_DEG), jnp.float32),
    mesh=_mesh,
    scratch_types=[
        pltpu.VMEM((DROW, DCH), jnp.int32),
        pltpu.VMEM((DCH,), jnp.float32),
        pltpu.VMEM((640,), jnp.float32),
        pltpu.VMEM_SHARED((NP_DEG,), jnp.float32),
    ],
)
def _sc_degree(dst_h, out_h, dstv, onesv, zv, acc):
    c = lax.axis_index("c")
    s = lax.axis_index("s")
    w = s * NC + c
    pltpu.sync_copy(dst_h.at[w], dstv)

    def fill_ones(i, _):
        onesv[pl.ds(i * 16, 16)] = jnp.ones((16,), jnp.float32)
        return 0
    lax.fori_loop(0, DCH // 16, fill_ones, 0)

    def fill_zero(i, _):
        zv[pl.ds(i * 16, 16)] = jnp.zeros((16,), jnp.float32)
        return 0
    lax.fori_loop(0, 640 // 16, fill_zero, 0)

    pltpu.sync_copy(zv, acc.at[pl.ds(s * 640, 640)])
    plsc.subcore_barrier()

    def body(r, _):
        pltpu.sync_copy(onesv, acc.at[dstv.at[r]], add=True)
        return 0
    lax.fori_loop(0, DROW, body, 0)

    plsc.subcore_barrier()
    pltpu.sync_copy(acc.at[pl.ds(s * 640, 640)], out_h.at[c, pl.ds(s * 640, 640)])


_HIMASK = jnp.int32(-65536)


@functools.partial(
    pl.kernel,
    out_type=jax.ShapeDtypeStruct((H * N,), jnp.float32),
    mesh=_mesh,
    compiler_params=pltpu.CompilerParams(needs_layout_passes=False),
    scratch_types=[
        [pltpu.VMEM((N,), jnp.int32) for _ in range(PK)],
        [pltpu.VMEM((N,), jnp.float32) for _ in range(FPW)],
        [pltpu.VMEM((CHK,), jnp.int32) for _ in range(2)],
        [pltpu.VMEM((CHK,), jnp.int32) for _ in range(2)],
        pltpu.SemaphoreType.DMA,
        pltpu.SemaphoreType.DMA,
    ],
)
def _sc_scatter(ysf_h, pkf_h, src_h, dst_h, accf_h, pkr, acr, sb, db, sem0, sem1):
    c = lax.axis_index("c")
    s = lax.axis_index("s")
    w = s * NC + c

    def load_idx(j, b, sem):
        pltpu.async_copy(src_h.at[pl.ds(j * CHK, CHK)], sb[b], sem)
        pltpu.async_copy(dst_h.at[pl.ds(j * CHK, CHK)], db[b], sem)

    def wait_idx(j, b, sem):
        pltpu.make_async_copy(src_h.at[pl.ds(j * CHK, CHK)], sb[b], sem).wait()
        pltpu.make_async_copy(dst_h.at[pl.ds(j * CHK, CHK)], db[b], sem).wait()

    def compute(b):
        def grp(kk, _):
            pend = []
            for u in range(UNROLL):
                k = kk * UNROLL + u
                s16 = sb[b][pl.ds(k * 16, 16)]
                d16 = db[b][pl.ds(k * 16, 16)]
                for t in range(PK):
                    pend.append((t, d16, plsc.load_gather(pkr[t], [s16])))
            for t, d16, v in pend:
                lo = plsc.bitcast(lax.shift_left(v, 16), jnp.float32)
                hi = plsc.bitcast(lax.bitwise_and(v, _HIMASK), jnp.float32)
                plsc.addupdate_scatter(acr[t], [d16], lo)
                plsc.addupdate_scatter(acr[PK + t], [d16], hi)
            return 0
        lax.fori_loop(0, GRP // UNROLL, grp, 0)

    for t in range(PK):
        pltpu.sync_copy(pkf_h.at[pl.ds((w * PK + t) * N, N)], pkr[t])
        pltpu.sync_copy(ysf_h.at[pl.ds((w * PK + t) * N, N)], acr[t])
        pltpu.sync_copy(ysf_h.at[pl.ds((H // 2 + w * PK + t) * N, N)], acr[PK + t])

    load_idx(0, 0, sem0)

    def outer(jj, _):
        j0 = 2 * jj
        load_idx(j0 + 1, 1, sem1)
        wait_idx(j0, 0, sem0)
        compute(0)

        @pl.when(jj < NCHK // 2 - 1)
        def _():
            load_idx(j0 + 2, 0, sem0)

        wait_idx(j0 + 1, 1, sem1)
        compute(1)
        return 0
    lax.fori_loop(0, NCHK // 2, outer, 0)

    for t in range(PK):
        pltpu.sync_copy(acr[t], accf_h.at[pl.ds((w * PK + t) * N, N)])
        pltpu.sync_copy(acr[PK + t], accf_h.at[pl.ds((H // 2 + w * PK + t) * N, N)])


def _pack_pairs(ys):
    lo = lax.bitcast_convert_type(
        ys[:H // 2, :].astype(jnp.bfloat16), jnp.uint16).astype(jnp.uint32)
    hi = lax.bitcast_convert_type(
        ys[H // 2:, :].astype(jnp.bfloat16), jnp.uint16).astype(jnp.uint32)
    return lax.bitcast_convert_type(lo | (hi << 16), jnp.int32)


def _k0_body(xt_ref, deg2_ref, w1t_ref, b1_ref, w2t_ref, b2_ref, w0t_ref,
             yst_ref, pk_ref, dinv_ref):
    deg = deg2_ref[0:1, :] + deg2_ref[1:2, :] + 1.0
    dinv = lax.rsqrt(deg)
    dinv_ref[...] = dinv
    h = jnp.maximum(
        jnp.dot(w1t_ref[...], xt_ref[...], preferred_element_type=jnp.float32)
        + b1_ref[...], 0.0)
    h = jnp.dot(w2t_ref[...], h, preferred_element_type=jnp.float32) + b2_ref[...]
    y = jnp.dot(w0t_ref[...], h, preferred_element_type=jnp.float32)
    ys = y * dinv
    yst_ref[...] = ys
    pk_ref[...] = _pack_pairs(ys)


_k0_call = pl.pallas_call(
    _k0_body,
    out_shape=[
        jax.ShapeDtypeStruct((H, N), jnp.float32),
        jax.ShapeDtypeStruct((H // 2, N), jnp.int32),
        jax.ShapeDtypeStruct((1, N), jnp.float32),
    ],
)


def _mid_body(at_ref, dinv_ref, bprev_ref, wnt_ref, yst_ref, pk_ref):
    dinv = dinv_ref[...]
    h = jnp.maximum(dinv * at_ref[...] + bprev_ref[...], 0.0)
    y = jnp.dot(wnt_ref[...], h, preferred_element_type=jnp.float32)
    ys = y * dinv
    yst_ref[...] = ys
    pk_ref[...] = _pack_pairs(ys)


_mid_call = pl.pallas_call(
    _mid_body,
    out_shape=[
        jax.ShapeDtypeStruct((H, N), jnp.float32),
        jax.ShapeDtypeStruct((H // 2, N), jnp.int32),
    ],
)


def _head_body(at_ref, dinv_ref, b2_ref,
               muW1t_ref, mub1_ref, muW2t_ref, mub2_ref,
               lvW1t_ref, lvb1_ref, lvW2t_ref, lvb2_ref,
               z_ref, mu_ref, lv_ref):
    h = jnp.maximum(dinv_ref[...] * at_ref[...] + b2_ref[...], 0.0)
    mean = jnp.sum(h, axis=1, keepdims=True) * (1.0 / N)
    mx = jnp.max(h, axis=1, keepdims=True)
    g = jnp.concatenate([mean, mx], axis=0)
    mu = jnp.maximum(
        jnp.dot(muW1t_ref[...], g, preferred_element_type=jnp.float32)
        + mub1_ref[...], 0.0)
    mu = jnp.dot(muW2t_ref[...], mu, preferred_element_type=jnp.float32) + mub2_ref[...]
    lv = jnp.maximum(
        jnp.dot(lvW1t_ref[...], g, preferred_element_type=jnp.float32)
        + lvb1_ref[...], 0.0)
    lv = jnp.dot(lvW2t_ref[...], lv, preferred_element_type=jnp.float32) + lvb2_ref[...]
    z_ref[...] = mu
    mu_ref[...] = mu
    lv_ref[...] = lv


_head_call = pl.pallas_call(
    _head_body,
    out_shape=[
        jax.ShapeDtypeStruct((ED, 1), jnp.float32),
        jax.ShapeDtypeStruct((ED, 1), jnp.float32),
        jax.ShapeDtypeStruct((ED, 1), jnp.float32),
    ],
)


def kernel(x, edge_index, enc_W1, enc_b1, enc_W2, enc_b2,
           conv_W0, conv_b0, conv_W1, conv_b1, conv_W2, conv_b2,
           mu_W1, mu_b1, mu_W2, mu_b2, lv_W1, lv_b1, lv_W2, lv_b2):
    src = edge_index[0]
    dst = edge_index[1]
    dstd = dst.reshape(NW, DROW, DCH)

    degp = _sc_degree(dstd)
    deg2 = degp[:, :N]

    yst, pk, dinv = _k0_call(x.T, deg2, enc_W1.T, enc_b1.reshape(H, 1),
                             enc_W2.T, enc_b2.reshape(H, 1), conv_W0.T)
    at = _sc_scatter(yst.reshape(H * N), pk.reshape(H // 2 * N), src, dst).reshape(H, N)
    yst, pk = _mid_call(at, dinv, conv_b0.reshape(H, 1), conv_W1.T)
    at = _sc_scatter(yst.reshape(H * N), pk.reshape(H // 2 * N), src, dst).reshape(H, N)
    yst, pk = _mid_call(at, dinv, conv_b1.reshape(H, 1), conv_W2.T)
    at = _sc_scatter(yst.reshape(H * N), pk.reshape(H // 2 * N), src, dst).reshape(H, N)
    z, mu, lv = _head_call(at, dinv, conv_b2.reshape(H, 1),
                           mu_W1.T, mu_b1.reshape(H, 1), mu_W2.T, mu_b2.reshape(ED, 1),
                           lv_W1.T, lv_b1.reshape(H, 1), lv_W2.T, lv_b2.reshape(ED, 1))
    return (z.reshape(1, ED), mu.reshape(1, ED), lv.reshape(1, ED))


# speedup vs baseline: 1.2744x; 1.0531x over previous
import functools

import jax
import jax.numpy as jnp
from jax import lax
from jax.experimental import pallas as pl
from jax.experimental.pallas import tpu as pltpu
from jax.experimental.pallas import tpu_sc as plsc

N = 10000
E = 320000
DF = 128
H = 256
ED = 128

NC = 2
NS = 16
NW = NC * NS
FPW = H // NW
PK = FPW // 2

CHK = 2000
NCHK = E // CHK
GRP = CHK // 16
UNROLL = 5

DCH = 80
DROW = E // NW // DCH
NP_DEG = 10240

_mesh = plsc.VectorSubcoreMesh(
    core_axis_name="c", subcore_axis_name="s", num_cores=NC, num_subcores=NS)


@functools.partial(
    pl.kernel,
    out_type=jax.ShapeDtypeStruct((NC, NP_DEG), jnp.float32),
    mesh=_mesh,
    scratch_types=[
        pltpu.VMEM((DROW, DCH), jnp.int32),
        pltpu.VMEM((DCH,), jnp.float32),
        pltpu.VMEM((640,), jnp.float32),
        pltpu.VMEM_SHARED((NP_DEG,), jnp.float32),
    ],
)
def _sc_degree(dst_h, out_h, dstv, onesv, zv, acc):
    c = lax.axis_index("c")
    s = lax.axis_index("s")
    w = s * NC + c
    pltpu.sync_copy(dst_h.at[w], dstv)

    def fill_ones(i, _):
        onesv[pl.ds(i * 16, 16)] = jnp.ones((16,), jnp.float32)
        return 0
    lax.fori_loop(0, DCH // 16, fill_ones, 0)

    def fill_zero(i, _):
        zv[pl.ds(i * 16, 16)] = jnp.zeros((16,), jnp.float32)
        return 0
    lax.fori_loop(0, 640 // 16, fill_zero, 0)

    pltpu.sync_copy(zv, acc.at[pl.ds(s * 640, 640)])
    plsc.subcore_barrier()

    def body(r, _):
        pltpu.sync_copy(onesv, acc.at[dstv.at[r]], add=True)
        return 0
    lax.fori_loop(0, DROW, body, 0)

    plsc.subcore_barrier()
    pltpu.sync_copy(acc.at[pl.ds(s * 640, 640)], out_h.at[c, pl.ds(s * 640, 640)])


_HIMASK = jnp.int32(-65536)


@functools.partial(
    pl.kernel,
    out_type=jax.ShapeDtypeStruct((H * N,), jnp.float32),
    mesh=_mesh,
    compiler_params=pltpu.CompilerParams(needs_layout_passes=False),
    scratch_types=[
        [pltpu.VMEM((N,), jnp.int32) for _ in range(PK)],
        [pltpu.VMEM((N,), jnp.float32) for _ in range(FPW)],
        [pltpu.VMEM((CHK,), jnp.int32) for _ in range(2)],
        [pltpu.VMEM((CHK,), jnp.int32) for _ in range(2)],
        pltpu.SemaphoreType.DMA,
        pltpu.SemaphoreType.DMA,
    ],
)
def _sc_scatter(pkf_h, src_h, dst_h, accf_h, pkr, acr, sb, db, sem0, sem1):
    c = lax.axis_index("c")
    s = lax.axis_index("s")
    w = s * NC + c

    def load_idx(j, b, sem):
        pltpu.async_copy(src_h.at[pl.ds(j * CHK, CHK)], sb[b], sem)
        pltpu.async_copy(dst_h.at[pl.ds(j * CHK, CHK)], db[b], sem)

    def wait_idx(j, b, sem):
        pltpu.make_async_copy(src_h.at[pl.ds(j * CHK, CHK)], sb[b], sem).wait()
        pltpu.make_async_copy(dst_h.at[pl.ds(j * CHK, CHK)], db[b], sem).wait()

    def compute(b):
        def grp(kk, _):
            pend = []
            for u in range(UNROLL):
                k = kk * UNROLL + u
                s16 = sb[b][pl.ds(k * 16, 16)]
                d16 = db[b][pl.ds(k * 16, 16)]
                for t in range(PK):
                    pend.append((t, d16, plsc.load_gather(pkr[t], [s16])))
            for t, d16, v in pend:
                lo = plsc.bitcast(lax.shift_left(v, 16), jnp.float32)
                hi = plsc.bitcast(lax.bitwise_and(v, _HIMASK), jnp.float32)
                plsc.addupdate_scatter(acr[t], [d16], lo)
                plsc.addupdate_scatter(acr[PK + t], [d16], hi)
            return 0
        lax.fori_loop(0, GRP // UNROLL, grp, 0)

    for t in range(PK):
        pltpu.sync_copy(pkf_h.at[pl.ds((w * PK + t) * N, N)], pkr[t])

    def initloop(i, _):
        for u in range(UNROLL):
            o = (i * UNROLL + u) * 16
            for t in range(PK):
                v = pkr[t][pl.ds(o, 16)]
                acr[t][pl.ds(o, 16)] = plsc.bitcast(
                    lax.shift_left(v, 16), jnp.float32)
                acr[PK + t][pl.ds(o, 16)] = plsc.bitcast(
                    lax.bitwise_and(v, _HIMASK), jnp.float32)
        return 0
    lax.fori_loop(0, N // 16 // UNROLL, initloop, 0)

    load_idx(0, 0, sem0)

    def outer(jj, _):
        j0 = 2 * jj
        load_idx(j0 + 1, 1, sem1)
        wait_idx(j0, 0, sem0)
        compute(0)

        @pl.when(jj < NCHK // 2 - 1)
        def _():
            load_idx(j0 + 2, 0, sem0)

        wait_idx(j0 + 1, 1, sem1)
        compute(1)
        return 0
    lax.fori_loop(0, NCHK // 2, outer, 0)

    for t in range(PK):
        pltpu.sync_copy(acr[t], accf_h.at[pl.ds((w * PK + t) * N, N)])
        pltpu.sync_copy(acr[PK + t], accf_h.at[pl.ds((H // 2 + w * PK + t) * N, N)])


def _pack_pairs(ys):
    lo = lax.bitcast_convert_type(
        ys[:H // 2, :].astype(jnp.bfloat16), jnp.uint16).astype(jnp.uint32)
    hi = lax.bitcast_convert_type(
        ys[H // 2:, :].astype(jnp.bfloat16), jnp.uint16).astype(jnp.uint32)
    return lax.bitcast_convert_type(lo | (hi << 16), jnp.int32)


def _k0_body(xt_ref, deg2_ref, w1t_ref, b1_ref, w2t_ref, b2_ref, w0t_ref,
             pk_ref, dinv_ref):
    deg = deg2_ref[0:1, :] + deg2_ref[1:2, :] + 1.0
    dinv = lax.rsqrt(deg)
    dinv_ref[...] = dinv
    h = jnp.maximum(
        jnp.dot(w1t_ref[...], xt_ref[...], preferred_element_type=jnp.float32)
        + b1_ref[...], 0.0)
    h = jnp.dot(w2t_ref[...], h, preferred_element_type=jnp.float32) + b2_ref[...]
    y = jnp.dot(w0t_ref[...], h, preferred_element_type=jnp.float32)
    pk_ref[...] = _pack_pairs(y * dinv)


_k0_call = pl.pallas_call(
    _k0_body,
    out_shape=[
        jax.ShapeDtypeStruct((H // 2, N), jnp.int32),
        jax.ShapeDtypeStruct((1, N), jnp.float32),
    ],
)


def _mid_body(at_ref, dinv_ref, bprev_ref, wnt_ref, pk_ref):
    dinv = dinv_ref[...]
    h = jnp.maximum(dinv * at_ref[...] + bprev_ref[...], 0.0)
    y = jnp.dot(wnt_ref[...], h, preferred_element_type=jnp.float32)
    pk_ref[...] = _pack_pairs(y * dinv)


_mid_call = pl.pallas_call(
    _mid_body,
    out_shape=jax.ShapeDtypeStruct((H // 2, N), jnp.int32),
)


def _head_body(at_ref, dinv_ref, b2_ref,
               muW1t_ref, mub1_ref, muW2t_ref, mub2_ref,
               lvW1t_ref, lvb1_ref, lvW2t_ref, lvb2_ref,
               z_ref, mu_ref, lv_ref):
    h = jnp.maximum(dinv_ref[...] * at_ref[...] + b2_ref[...], 0.0)
    mean = jnp.sum(h, axis=1, keepdims=True) * (1.0 / N)
    mx = jnp.max(h, axis=1, keepdims=True)
    g = jnp.concatenate([mean, mx], axis=0)
    mu = jnp.maximum(
        jnp.dot(muW1t_ref[...], g, preferred_element_type=jnp.float32)
        + mub1_ref[...], 0.0)
    mu = jnp.dot(muW2t_ref[...], mu, preferred_element_type=jnp.float32) + mub2_ref[...]
    lv = jnp.maximum(
        jnp.dot(lvW1t_ref[...], g, preferred_element_type=jnp.float32)
        + lvb1_ref[...], 0.0)
    lv = jnp.dot(lvW2t_ref[...], lv, preferred_element_type=jnp.float32) + lvb2_ref[...]
    z_ref[...] = mu
    mu_ref[...] = mu
    lv_ref[...] = lv


_head_call = pl.pallas_call(
    _head_body,
    out_shape=[
        jax.ShapeDtypeStruct((ED, 1), jnp.float32),
        jax.ShapeDtypeStruct((ED, 1), jnp.float32),
        jax.ShapeDtypeStruct((ED, 1), jnp.float32),
    ],
)


def kernel(x, edge_index, enc_W1, enc_b1, enc_W2, enc_b2,
           conv_W0, conv_b0, conv_W1, conv_b1, conv_W2, conv_b2,
           mu_W1, mu_b1, mu_W2, mu_b2, lv_W1, lv_b1, lv_W2, lv_b2):
    src = edge_index[0]
    dst = edge_index[1]
    dstd = dst.reshape(NW, DROW, DCH)

    degp = _sc_degree(dstd)
    deg2 = degp[:, :N]

    pk, dinv = _k0_call(x.T, deg2, enc_W1.T, enc_b1.reshape(H, 1),
                        enc_W2.T, enc_b2.reshape(H, 1), conv_W0.T)
    at = _sc_scatter(pk.reshape(H // 2 * N), src, dst).reshape(H, N)
    pk = _mid_call(at, dinv, conv_b0.reshape(H, 1), conv_W1.T)
    at = _sc_scatter(pk.reshape(H // 2 * N), src, dst).reshape(H, N)
    pk = _mid_call(at, dinv, conv_b1.reshape(H, 1), conv_W2.T)
    at = _sc_scatter(pk.reshape(H // 2 * N), src, dst).reshape(H, N)
    z, mu, lv = _head_call(at, dinv, conv_b2.reshape(H, 1),
                           mu_W1.T, mu_b1.reshape(H, 1), mu_W2.T, mu_b2.reshape(ED, 1),
                           lv_W1.T, lv_b1.reshape(H, 1), lv_W2.T, lv_b2.reshape(ED, 1))
    return (z.reshape(1, ED), mu.reshape(1, ED), lv.reshape(1, ED))


# speedup vs baseline: 1.2793x; 1.0038x over previous
import functools

import jax
import jax.numpy as jnp
from jax import lax
from jax.experimental import pallas as pl
from jax.experimental.pallas import tpu as pltpu
from jax.experimental.pallas import tpu_sc as plsc

N = 10000
NP = 10112
E = 320000
DF = 128
H = 256
ED = 128

NC = 2
NS = 16
NW = NC * NS
FPW = H // NW
PK = FPW // 2

CHK = 2000
NCHK = E // CHK
GRP = CHK // 16
UNROLL = 5

DCH = 80
DROW = E // NW // DCH
NP_DEG = 10240

_mesh = plsc.VectorSubcoreMesh(
    core_axis_name="c", subcore_axis_name="s", num_cores=NC, num_subcores=NS)


@functools.partial(
    pl.kernel,
    out_type=jax.ShapeDtypeStruct((NC, NP_DEG), jnp.float32),
    mesh=_mesh,
    scratch_types=[
        pltpu.VMEM((DROW, DCH), jnp.int32),
        pltpu.VMEM((DCH,), jnp.float32),
        pltpu.VMEM((640,), jnp.float32),
        pltpu.VMEM_SHARED((NP_DEG,), jnp.float32),
    ],
)
def _sc_degree(dst_h, out_h, dstv, onesv, zv, acc):
    c = lax.axis_index("c")
    s = lax.axis_index("s")
    w = s * NC + c
    pltpu.sync_copy(dst_h.at[w], dstv)

    def fill_ones(i, _):
        onesv[pl.ds(i * 16, 16)] = jnp.ones((16,), jnp.float32)
        return 0
    lax.fori_loop(0, DCH // 16, fill_ones, 0)

    def fill_zero(i, _):
        zv[pl.ds(i * 16, 16)] = jnp.zeros((16,), jnp.float32)
        return 0
    lax.fori_loop(0, 640 // 16, fill_zero, 0)

    pltpu.sync_copy(zv, acc.at[pl.ds(s * 640, 640)])
    plsc.subcore_barrier()

    def body(r, _):
        pltpu.sync_copy(onesv, acc.at[dstv.at[r]], add=True)
        return 0
    lax.fori_loop(0, DROW, body, 0)

    plsc.subcore_barrier()
    pltpu.sync_copy(acc.at[pl.ds(s * 640, 640)], out_h.at[c, pl.ds(s * 640, 640)])


_HIMASK = jnp.int32(-65536)


@functools.partial(
    pl.kernel,
    out_type=jax.ShapeDtypeStruct((H * NP,), jnp.float32),
    mesh=_mesh,
    compiler_params=pltpu.CompilerParams(needs_layout_passes=False),
    scratch_types=[
        [pltpu.VMEM((NP,), jnp.int32) for _ in range(PK)],
        [pltpu.VMEM((NP,), jnp.float32) for _ in range(FPW)],
        [pltpu.VMEM((CHK,), jnp.int32) for _ in range(2)],
        [pltpu.VMEM((CHK,), jnp.int32) for _ in range(2)],
        pltpu.SemaphoreType.DMA,
        pltpu.SemaphoreType.DMA,
    ],
)
def _sc_scatter(pkf_h, src_h, dst_h, accf_h, pkr, acr, sb, db, sem0, sem1):
    c = lax.axis_index("c")
    s = lax.axis_index("s")
    w = s * NC + c

    def load_idx(j, b, sem):
        pltpu.async_copy(src_h.at[pl.ds(j * CHK, CHK)], sb[b], sem)
        pltpu.async_copy(dst_h.at[pl.ds(j * CHK, CHK)], db[b], sem)

    def wait_idx(j, b, sem):
        pltpu.make_async_copy(src_h.at[pl.ds(j * CHK, CHK)], sb[b], sem).wait()
        pltpu.make_async_copy(dst_h.at[pl.ds(j * CHK, CHK)], db[b], sem).wait()

    def compute(b):
        def grp(kk, _):
            pend = []
            for u in range(UNROLL):
                k = kk * UNROLL + u
                s16 = sb[b][pl.ds(k * 16, 16)]
                d16 = db[b][pl.ds(k * 16, 16)]
                for t in range(PK):
                    pend.append((t, d16, plsc.load_gather(pkr[t], [s16])))
            for t, d16, v in pend:
                lo = plsc.bitcast(lax.shift_left(v, 16), jnp.float32)
                hi = plsc.bitcast(lax.bitwise_and(v, _HIMASK), jnp.float32)
                plsc.addupdate_scatter(acr[t], [d16], lo)
                plsc.addupdate_scatter(acr[PK + t], [d16], hi)
            return 0
        lax.fori_loop(0, GRP // UNROLL, grp, 0)

    for t in range(PK):
        pltpu.sync_copy(pkf_h.at[pl.ds((w * PK + t) * NP, NP)], pkr[t])

    def initloop(i, _):
        for u in range(8):
            o = (i * 8 + u) * 16
            for t in range(PK):
                v = pkr[t][pl.ds(o, 16)]
                acr[t][pl.ds(o, 16)] = plsc.bitcast(
                    lax.shift_left(v, 16), jnp.float32)
                acr[PK + t][pl.ds(o, 16)] = plsc.bitcast(
                    lax.bitwise_and(v, _HIMASK), jnp.float32)
        return 0
    lax.fori_loop(0, NP // 16 // 8, initloop, 0)

    load_idx(0, 0, sem0)

    def outer(jj, _):
        j0 = 2 * jj
        load_idx(j0 + 1, 1, sem1)
        wait_idx(j0, 0, sem0)
        compute(0)

        @pl.when(jj < NCHK // 2 - 1)
        def _():
            load_idx(j0 + 2, 0, sem0)

        wait_idx(j0 + 1, 1, sem1)
        compute(1)
        return 0
    lax.fori_loop(0, NCHK // 2, outer, 0)

    for t in range(PK):
        pltpu.sync_copy(acr[t], accf_h.at[pl.ds((w * PK + t) * NP, NP)])
        pltpu.sync_copy(acr[PK + t],
                        accf_h.at[pl.ds((H // 2 + w * PK + t) * NP, NP)])


def _pack_pairs(ys):
    lo = lax.bitcast_convert_type(
        ys[:H // 2, :].astype(jnp.bfloat16), jnp.uint16).astype(jnp.uint32)
    hi = lax.bitcast_convert_type(
        ys[H // 2:, :].astype(jnp.bfloat16), jnp.uint16).astype(jnp.uint32)
    return lax.bitcast_convert_type(lo | (hi << 16), jnp.int32)


def _k0_body(xt_ref, deg2_ref, w1t_ref, b1_ref, w2t_ref, b2_ref, w0t_ref,
             pk_ref, dinv_ref):
    deg = deg2_ref[0:1, :] + deg2_ref[1:2, :] + 1.0
    dinv = lax.rsqrt(deg)
    dinv_ref[...] = dinv
    h = jnp.maximum(
        jnp.dot(w1t_ref[...], xt_ref[...], preferred_element_type=jnp.float32)
        + b1_ref[...], 0.0)
    h = jnp.dot(w2t_ref[...], h, preferred_element_type=jnp.float32) + b2_ref[...]
    y = jnp.dot(w0t_ref[...], h, preferred_element_type=jnp.float32)
    pk_ref[...] = _pack_pairs(y * dinv)


_k0_call = pl.pallas_call(
    _k0_body,
    out_shape=[
        jax.ShapeDtypeStruct((H // 2, NP), jnp.int32),
        jax.ShapeDtypeStruct((1, NP), jnp.float32),
    ],
)


def _mid_body(at_ref, dinv_ref, bprev_ref, wnt_ref, pk_ref):
    dinv = dinv_ref[...]
    h = jnp.maximum(dinv * at_ref[...] + bprev_ref[...], 0.0)
    y = jnp.dot(wnt_ref[...], h, preferred_element_type=jnp.float32)
    pk_ref[...] = _pack_pairs(y * dinv)


_mid_call = pl.pallas_call(
    _mid_body,
    out_shape=jax.ShapeDtypeStruct((H // 2, NP), jnp.int32),
)


def _head_body(at_ref, dinv_ref, b2_ref,
               muW1t_ref, mub1_ref, muW2t_ref, mub2_ref,
               lvW1t_ref, lvb1_ref, lvW2t_ref, lvb2_ref,
               z_ref, mu_ref, lv_ref):
    h = jnp.maximum(dinv_ref[...] * at_ref[...] + b2_ref[...], 0.0)
    lane = lax.broadcasted_iota(jnp.int32, (1, NP), 1)
    valid = lane < N
    mean = jnp.sum(jnp.where(valid, h, 0.0), axis=1, keepdims=True) * (1.0 / N)
    mx = jnp.max(jnp.where(valid, h, -jnp.inf), axis=1, keepdims=True)
    g = jnp.concatenate([mean, mx], axis=0)
    mu = jnp.maximum(
        jnp.dot(muW1t_ref[...], g, preferred_element_type=jnp.float32)
        + mub1_ref[...], 0.0)
    mu = jnp.dot(muW2t_ref[...], mu, preferred_element_type=jnp.float32) + mub2_ref[...]
    lv = jnp.maximum(
        jnp.dot(lvW1t_ref[...], g, preferred_element_type=jnp.float32)
        + lvb1_ref[...], 0.0)
    lv = jnp.dot(lvW2t_ref[...], lv, preferred_element_type=jnp.float32) + lvb2_ref[...]
    z_ref[...] = mu
    mu_ref[...] = mu
    lv_ref[...] = lv


_head_call = pl.pallas_call(
    _head_body,
    out_shape=[
        jax.ShapeDtypeStruct((ED, 1), jnp.float32),
        jax.ShapeDtypeStruct((ED, 1), jnp.float32),
        jax.ShapeDtypeStruct((ED, 1), jnp.float32),
    ],
)


def kernel(x, edge_index, enc_W1, enc_b1, enc_W2, enc_b2,
           conv_W0, conv_b0, conv_W1, conv_b1, conv_W2, conv_b2,
           mu_W1, mu_b1, mu_W2, mu_b2, lv_W1, lv_b1, lv_W2, lv_b2):
    src = edge_index[0]
    dst = edge_index[1]
    dstd = dst.reshape(NW, DROW, DCH)

    degp = _sc_degree(dstd)
    deg2 = degp[:, :NP]

    xt = jnp.pad(x.T, ((0, 0), (0, NP - N)))
    pk, dinv = _k0_call(xt, deg2, enc_W1.T, enc_b1.reshape(H, 1),
                        enc_W2.T, enc_b2.reshape(H, 1), conv_W0.T)
    at = _sc_scatter(pk.reshape(H // 2 * NP), src, dst).reshape(H, NP)
    pk = _mid_call(at, dinv, conv_b0.reshape(H, 1), conv_W1.T)
    at = _sc_scatter(pk.reshape(H // 2 * NP), src, dst).reshape(H, NP)
    pk = _mid_call(at, dinv, conv_b1.reshape(H, 1), conv_W2.T)
    at = _sc_scatter(pk.reshape(H // 2 * NP), src, dst).reshape(H, NP)
    z, mu, lv = _head_call(at, dinv, conv_b2.reshape(H, 1),
                           mu_W1.T, mu_b1.reshape(H, 1), mu_W2.T, mu_b2.reshape(ED, 1),
                           lv_W1.T, lv_b1.reshape(H, 1), lv_W2.T, lv_b2.reshape(ED, 1))
    return (z.reshape(1, ED), mu.reshape(1, ED), lv.reshape(1, ED))
